# Initial kernel scaffold; baseline (speedup 1.0000x reference)
import jax
import jax.numpy as jnp
from jax.experimental import pallas as pl


def kernel(logits, ids, u):
    raise NotImplementedError("write your pallas kernel here")


# speedup vs baseline: 2643.5626x; 2643.5626x over previous
import jax
import jax.numpy as jnp
from jax import lax
from jax.experimental import pallas as pl
from jax.experimental.pallas import tpu as pltpu
from jax.experimental.pallas import tpu_sc as plsc

N_BUCKETS = 8192
BS = 1048576
L = 16
NW = 32
PER_W = BS // NW
CHUNK = 8192
N_CHUNKS = PER_W // CHUNK
INV_N = 1.0 / N_BUCKETS


def _height_body(logits_ref, h_ref):
    x = logits_ref[...]
    m = jnp.max(x)
    e = jnp.exp(x - m)
    h_ref[...] = e * (N_BUCKETS / jnp.sum(e))


def _sampler_body(h_hbm, ids_hbm, u_hbm, t_hbm, p_hbm, h_v, ids_v, u_v, t_v, p_v):
    wid = lax.axis_index("s") * 2 + lax.axis_index("c")
    base = wid * PER_W
    pltpu.sync_copy(h_hbm, h_v)

    def chunk_body(ci, carry):
        off = base + ci * CHUNK
        pltpu.sync_copy(ids_hbm.at[pl.ds(off, CHUNK)], ids_v)
        pltpu.sync_copy(u_hbm.at[pl.ds(off, CHUNK)], u_v)

        def body(i, c):
            sl = pl.ds(i * L, L)
            s = ids_v[sl].astype(jnp.float32) + u_v[sl]
            t_v[sl] = s * INV_N
            idx = jnp.minimum(s.astype(jnp.int32), N_BUCKETS - 1)
            p_v[sl] = plsc.load_gather(h_v, [idx])
            return c

        lax.fori_loop(0, CHUNK // L, body, 0, unroll=4)
        pltpu.sync_copy(t_v, t_hbm.at[pl.ds(off, CHUNK)])
        pltpu.sync_copy(p_v, p_hbm.at[pl.ds(off, CHUNK)])
        return carry

    lax.fori_loop(0, N_CHUNKS, chunk_body, 0)


def kernel(logits, ids, u):
    h = pl.pallas_call(
        _height_body,
        out_shape=jax.ShapeDtypeStruct((8, N_BUCKETS // 8), jnp.float32),
    )(logits.reshape(8, N_BUCKETS // 8)).reshape(N_BUCKETS)

    sampler = pl.kernel(
        _sampler_body,
        out_type=[jax.ShapeDtypeStruct((BS,), jnp.float32),
                  jax.ShapeDtypeStruct((BS,), jnp.float32)],
        mesh=plsc.VectorSubcoreMesh(core_axis_name="c", subcore_axis_name="s"),
        compiler_params=pltpu.CompilerParams(needs_layout_passes=False),
        scratch_types=[
            pltpu.VMEM((N_BUCKETS,), jnp.float32),
            pltpu.VMEM((CHUNK,), jnp.int32),
            pltpu.VMEM((CHUNK,), jnp.float32),
            pltpu.VMEM((CHUNK,), jnp.float32),
            pltpu.VMEM((CHUNK,), jnp.float32),
        ],
    )
    t, p = sampler(h, ids, u)
    return (t[:, None], p)


# speedup vs baseline: 5536.4635x; 2.0943x over previous
import jax
import jax.numpy as jnp
from jax import lax
from jax.experimental import pallas as pl
from jax.experimental.pallas import tpu as pltpu
from jax.experimental.pallas import tpu_sc as plsc

N_BUCKETS = 8192
BS = 1048576
L = 16
NW = 32
PER_W = BS // NW
CHUNK = 8192
N_CHUNKS = PER_W // CHUNK
INV_N = 1.0 / N_BUCKETS


def _height_body(logits_ref, h_ref):
    x = logits_ref[...]
    m = jnp.max(x)
    e = jnp.exp(x - m)
    h_ref[...] = e * (N_BUCKETS / jnp.sum(e))


def _sampler_body(h_hbm, ids_hbm, u_hbm, t_hbm, p_hbm,
                  h_v, ids_v0, ids_v1, u_v0, u_v1, t_v0, t_v1, p_v0, p_v1,
                  sem_h, in_sem0, in_sem1, out_sem0, out_sem1):
    wid = lax.axis_index("s") * 2 + lax.axis_index("c")
    base = wid * PER_W
    ids_bufs = (ids_v0, ids_v1)
    u_bufs = (u_v0, u_v1)
    t_bufs = (t_v0, t_v1)
    p_bufs = (p_v0, p_v1)
    in_sems = (in_sem0, in_sem1)
    out_sems = (out_sem0, out_sem1)

    h_cp = pltpu.async_copy(h_hbm, h_v, sem_h)

    def start_in(ci):
        b = ci % 2
        off = base + ci * CHUNK
        return (pltpu.async_copy(ids_hbm.at[pl.ds(off, CHUNK)], ids_bufs[b], in_sems[b]),
                pltpu.async_copy(u_hbm.at[pl.ds(off, CHUNK)], u_bufs[b], in_sems[b]))

    in_cp = start_in(0)
    out_cp = [None, None]
    for ci in range(N_CHUNKS):
        b = ci % 2
        next_in = start_in(ci + 1) if ci + 1 < N_CHUNKS else None
        for cp in in_cp:
            cp.wait()
        if ci == 0:
            h_cp.wait()
        if out_cp[b] is not None:
            for cp in out_cp[b]:
                cp.wait()
        ids_b, u_b, t_b, p_b = ids_bufs[b], u_bufs[b], t_bufs[b], p_bufs[b]

        @plsc.parallel_loop(0, CHUNK, step=L, unroll=8)
        def _compute(i):
            s = ids_b[pl.ds(i, L)].astype(jnp.float32) + u_b[pl.ds(i, L)]
            t_b[pl.ds(i, L)] = s * INV_N
            idx = jnp.minimum(s.astype(jnp.int32), N_BUCKETS - 1)
            p_b[pl.ds(i, L)] = plsc.load_gather(h_v, [idx])

        off = base + ci * CHUNK
        out_cp[b] = (pltpu.async_copy(t_b, t_hbm.at[pl.ds(off, CHUNK)], out_sems[b]),
                     pltpu.async_copy(p_b, p_hbm.at[pl.ds(off, CHUNK)], out_sems[b]))
        if next_in is not None:
            in_cp = next_in
    for pair in out_cp:
        if pair is not None:
            for cp in pair:
                cp.wait()


def kernel(logits, ids, u):
    h = pl.pallas_call(
        _height_body,
        out_shape=jax.ShapeDtypeStruct((8, N_BUCKETS // 8), jnp.float32),
    )(logits.reshape(8, N_BUCKETS // 8)).reshape(N_BUCKETS)

    sampler = pl.kernel(
        _sampler_body,
        out_type=[jax.ShapeDtypeStruct((BS,), jnp.float32),
                  jax.ShapeDtypeStruct((BS,), jnp.float32)],
        mesh=plsc.VectorSubcoreMesh(core_axis_name="c", subcore_axis_name="s"),
        compiler_params=pltpu.CompilerParams(needs_layout_passes=False),
        scratch_types=[
            pltpu.VMEM((N_BUCKETS,), jnp.float32),
            pltpu.VMEM((CHUNK,), jnp.int32),
            pltpu.VMEM((CHUNK,), jnp.int32),
            pltpu.VMEM((CHUNK,), jnp.float32),
            pltpu.VMEM((CHUNK,), jnp.float32),
            pltpu.VMEM((CHUNK,), jnp.float32),
            pltpu.VMEM((CHUNK,), jnp.float32),
            pltpu.VMEM((CHUNK,), jnp.float32),
            pltpu.VMEM((CHUNK,), jnp.float32),
            pltpu.SemaphoreType.DMA,
            pltpu.SemaphoreType.DMA,
            pltpu.SemaphoreType.DMA,
            pltpu.SemaphoreType.DMA,
            pltpu.SemaphoreType.DMA,
        ],
    )
    t, p = sampler(h, ids, u)
    return (t[:, None], p)
